# baseline (speedup 1.0000x reference)
import functools

import jax
import jax.numpy as jnp
from jax import lax
from jax.experimental import pallas as pl
from jax.experimental.pallas import tpu as pltpu
from jax.experimental.pallas import tpu_sc as plsc

_B = 16384
_D = 64
_NW = 32
_BPW = _B // _NW
_RING = 8
_EMBW = 128


def _read_scalar(ref, j):
    v = plsc.load_gather(ref, [jnp.zeros((16,), jnp.int32) + j])
    return v[0]


def _gather_phase(tab_ref, sids_ref, emb_ref, wid,
                  ids_v, rr_v, blocks_v, starts_v, ring_v, echunk_v, sems):
    base = wid * _BPW
    pltpu.sync_copy(sids_ref.at[pl.ds(base, _BPW)], ids_v)

    lane = lax.iota(jnp.int32, 16)

    total = jnp.int32(0)
    for ch in range(_BPW // 16):
        pos = ch * 16 + lane
        idv = ids_v[pl.ds(ch * 16, 16)]
        cv = idv >> 7
        rr_v[pl.ds(ch * 16, 16)] = idv & 127
        prev_blk = plsc.load_gather(ids_v, [jnp.maximum(pos - 1, 0)]) >> 7
        m = (cv != prev_blk) | (pos == 0)
        plsc.store_compressed(blocks_v.at[pl.ds(total, 16)], cv, mask=m)
        plsc.store_compressed(starts_v.at[pl.ds(total, 16)], pos, mask=m)
        total = total + plsc.all_reduce_population_count(m)[0]
    nblocks = total
    plsc.store_compressed(starts_v.at[pl.ds(nblocks, 16)],
                          jnp.zeros((16,), jnp.int32) + _BPW,
                          mask=lane == 0)

    def fire(j, bank, q):
        @pl.when(j < nblocks)
        def _():
            cb = _read_scalar(blocks_v, j)
            slot = bank * 4 + q
            pltpu.async_copy(
                tab_ref.at[:, pl.ds(cb * 128, 128)],
                ring_v.at[pl.ds(slot * 64, 64), :],
                sems[slot])

    def extract(j, bank, q):
        @pl.when(j < nblocks)
        def _():
            slot = bank * 4 + q
            pltpu.make_async_copy(
                tab_ref.at[:, pl.ds(0, 128)],
                ring_v.at[pl.ds(slot * 64, 64), :],
                sems[slot]).wait()
            s0 = _read_scalar(starts_v, j)
            s1 = _read_scalar(starts_v, j + 1)

            def id_body(i, carry):
                rr_i = _read_scalar(rr_v, i)
                li = i & 15
                col = jnp.zeros((16,), jnp.int32) + rr_i
                for cc in range(4):
                    rows = slot * 64 + cc * 16 + lane
                    g = plsc.load_gather(ring_v, [rows, col])
                    echunk_v[pl.ds(li * _EMBW + cc * 16, 16)] = g

                @pl.when(li == 15)
                def _flush():
                    row0 = base + i - 15
                    pltpu.sync_copy(
                        echunk_v,
                        emb_ref.at[pl.ds(row0 * _EMBW, 16 * _EMBW)])
                return carry

            lax.fori_loop(s0, s1, id_body, 0)

    ngroups = (nblocks + 3) >> 2
    for g in range(2):
        for q in range(4):
            fire(g * 4 + q, g, q)

    def pair_body(h, carry):
        g0 = 2 * h
        for q in range(4):
            extract(g0 * 4 + q, 0, q)
        for q in range(4):
            fire((g0 + 2) * 4 + q, 0, q)
        for q in range(4):
            extract((g0 + 1) * 4 + q, 1, q)
        for q in range(4):
            fire((g0 + 3) * 4 + q, 1, q)
        return carry

    lax.fori_loop(0, (ngroups + 1) >> 1, pair_body, 0)


def _k1_body(us_hbm, utT_hbm, uemb_hbm,
             ids_v, rr_v, blocks_v, starts_v, ring_v, echunk_v, *sems):
    wid = lax.axis_index("s") * 2 + lax.axis_index("c")
    _gather_phase(utT_hbm, us_hbm, uemb_hbm, wid,
                  ids_v, rr_v, blocks_v, starts_v, ring_v, echunk_v, sems)


def _k2_body(uemb_hbm, vemb_hbm, iu_hbm, iv_hbm, out_hbm,
             iu_v, iv_v, u_rows, v_rows, partial_v, scores_v,
             sem_u, sem_v):
    wid = lax.axis_index("s") * 2 + lax.axis_index("c")
    base = wid * _BPW
    lane = lax.iota(jnp.int32, 16)
    nk = _BPW // 128

    def stage_idx(k):
        row0 = base + k * 128
        b = k & 1
        pltpu.sync_copy(iu_hbm.at[pl.ds(row0, 128)], iu_v.at[b])
        pltpu.sync_copy(iv_hbm.at[pl.ds(row0, 128)], iv_v.at[b])

    def fire_rows(k):
        b = k & 1
        pltpu.async_copy(uemb_hbm.at[iu_v.at[b]], u_rows.at[b], sem_u)
        pltpu.async_copy(vemb_hbm.at[iv_v.at[b]], v_rows.at[b], sem_v)

    def wait_rows(k):
        b = k & 1
        pltpu.make_async_copy(uemb_hbm.at[iu_v.at[b]], u_rows.at[b],
                              sem_u).wait()
        pltpu.make_async_copy(vemb_hbm.at[iv_v.at[b]], v_rows.at[b],
                              sem_v).wait()

    def compute(k):
        b = k & 1
        row0 = base + k * 128

        def row_body(r, carry):
            acc = jnp.zeros((16,), jnp.float32)
            for cc in range(_D // 16):
                gu = u_rows[b, r, pl.ds(cc * 16, 16)]
                gv = v_rows[b, r, pl.ds(cc * 16, 16)]
                acc = acc + gu * gv
            partial_v[pl.ds(r * 16, 16)] = acc
            return carry

        lax.fori_loop(0, 128, row_body, 0)

        def tile_body(t, carry):
            rowbase = (t * 16 + lane) * 16
            acc = jnp.zeros((16,), jnp.float32)
            for l in range(16):
                acc = acc + plsc.load_gather(partial_v, [rowbase + l])
            scores_v[pl.ds(t * 16, 16)] = acc
            return carry

        lax.fori_loop(0, 8, tile_body, 0)
        pltpu.sync_copy(scores_v, out_hbm.at[pl.ds(row0, 128)])

    stage_idx(0)
    fire_rows(0)
    for k in range(nk):
        if k + 1 < nk:
            stage_idx(k + 1)
            fire_rows(k + 1)
        wait_rows(k)
        compute(k)


def kernel(user_ids, item_ids, user_table, item_table):
    mesh = plsc.VectorSubcoreMesh(core_axis_name="c", subcore_axis_name="s")
    cp = pltpu.CompilerParams(
        needs_layout_passes=False, use_tc_tiling_on_sc=True)

    ui = user_ids.astype(jnp.int32)
    ii = item_ids.astype(jnp.int32)
    iota_b = jnp.arange(_B, dtype=jnp.int32)
    us, order_u = lax.sort_key_val(ui, iota_b)
    vs, order_v = lax.sort_key_val(ii, iota_b)
    inv_u = jnp.zeros((_B,), jnp.int32).at[order_u].set(iota_b)
    inv_v = jnp.zeros((_B,), jnp.int32).at[order_v].set(iota_b)

    k1 = functools.partial(
        pl.kernel, mesh=mesh, compiler_params=cp,
        out_type=jax.ShapeDtypeStruct((_B * _EMBW,), jnp.float32),
        scratch_types=[
            pltpu.VMEM((_BPW,), jnp.int32),
            pltpu.VMEM((_BPW,), jnp.int32),
            pltpu.VMEM((_BPW + 48,), jnp.int32),
            pltpu.VMEM((_BPW + 48,), jnp.int32),
            pltpu.VMEM((_RING * 64, 128), jnp.float32),
            pltpu.VMEM((16 * _EMBW,), jnp.float32),
        ] + [pltpu.SemaphoreType.DMA] * _RING,
    )(_k1_body)
    uemb = k1(us, user_table.T)
    vemb = k1(vs, item_table.T)

    uemb2 = uemb.reshape(_B, _EMBW)
    vemb2 = vemb.reshape(_B, _EMBW)

    k2 = functools.partial(
        pl.kernel, mesh=mesh, compiler_params=cp,
        out_type=jax.ShapeDtypeStruct((_B,), jnp.float32),
        scratch_types=[
            pltpu.VMEM((2, 128), jnp.int32),
            pltpu.VMEM((2, 128), jnp.int32),
            pltpu.VMEM((2, 128, _EMBW), jnp.float32),
            pltpu.VMEM((2, 128, _EMBW), jnp.float32),
            pltpu.VMEM((128 * 16,), jnp.float32),
            pltpu.VMEM((128,), jnp.float32),
            pltpu.SemaphoreType.DMA,
            pltpu.SemaphoreType.DMA,
        ],
    )(_k2_body)
    return k2(uemb2, vemb2, inv_u, inv_v)


# speedup vs baseline: 1.0074x; 1.0074x over previous
import functools

import jax
import jax.numpy as jnp
from jax import lax
from jax.experimental import pallas as pl
from jax.experimental.pallas import tpu as pltpu
from jax.experimental.pallas import tpu_sc as plsc

_B = 16384
_D = 64
_NW = 32
_BPW = _B // _NW
_RING = 8
_EMBW = 128


def _read_scalar(ref, j):
    v = plsc.load_gather(ref, [jnp.zeros((16,), jnp.int32) + j])
    return v[0]


def _gather_phase(tab_ref, sids_ref, emb_ref, wid,
                  ids_v, rr_v, blocks_v, starts_v, ring_v, echunk_v, sems):
    base = wid * _BPW
    pltpu.sync_copy(sids_ref.at[pl.ds(base, _BPW)], ids_v)

    lane = lax.iota(jnp.int32, 16)

    total = jnp.int32(0)
    for ch in range(_BPW // 16):
        pos = ch * 16 + lane
        idv = ids_v[pl.ds(ch * 16, 16)]
        cv = idv >> 7
        rr_v[pl.ds(ch * 16, 16)] = idv & 127
        prev_blk = plsc.load_gather(ids_v, [jnp.maximum(pos - 1, 0)]) >> 7
        m = (cv != prev_blk) | (pos == 0)
        plsc.store_compressed(blocks_v.at[pl.ds(total, 16)], cv, mask=m)
        plsc.store_compressed(starts_v.at[pl.ds(total, 16)], pos, mask=m)
        total = total + plsc.all_reduce_population_count(m)[0]
    nblocks = total
    plsc.store_compressed(starts_v.at[pl.ds(nblocks, 16)],
                          jnp.zeros((16,), jnp.int32) + _BPW,
                          mask=lane == 0)

    def fire(j, bank, q):
        @pl.when(j < nblocks)
        def _():
            cb = _read_scalar(blocks_v, j)
            slot = bank * 4 + q
            pltpu.async_copy(
                tab_ref.at[:, pl.ds(cb * 128, 128)],
                ring_v.at[pl.ds(slot * 64, 64), :],
                sems[slot])

    def extract(j, bank, q):
        @pl.when(j < nblocks)
        def _():
            slot = bank * 4 + q
            pltpu.make_async_copy(
                tab_ref.at[:, pl.ds(0, 128)],
                ring_v.at[pl.ds(slot * 64, 64), :],
                sems[slot]).wait()
            s0 = _read_scalar(starts_v, j)
            s1 = _read_scalar(starts_v, j + 1)

            def id_body(i, carry):
                rr_i = _read_scalar(rr_v, i)
                li = i & 15
                col = jnp.zeros((16,), jnp.int32) + rr_i
                for cc in range(4):
                    rows = slot * 64 + cc * 16 + lane
                    g = plsc.load_gather(ring_v, [rows, col])
                    echunk_v[pl.ds(li * _EMBW + cc * 16, 16)] = g

                @pl.when(li == 15)
                def _flush():
                    row0 = base + i - 15
                    pltpu.sync_copy(
                        echunk_v,
                        emb_ref.at[pl.ds(row0 * _EMBW, 16 * _EMBW)])
                return carry

            lax.fori_loop(s0, s1, id_body, 0)

    ngroups = (nblocks + 3) >> 2
    for g in range(2):
        for q in range(4):
            fire(g * 4 + q, g, q)

    def pair_body(h, carry):
        g0 = 2 * h
        for q in range(4):
            extract(g0 * 4 + q, 0, q)
        for q in range(4):
            fire((g0 + 2) * 4 + q, 0, q)
        for q in range(4):
            extract((g0 + 1) * 4 + q, 1, q)
        for q in range(4):
            fire((g0 + 3) * 4 + q, 1, q)
        return carry

    lax.fori_loop(0, (ngroups + 1) >> 1, pair_body, 0)


def _k1_body(us_hbm, vs_hbm, utT_hbm, itT_hbm, uemb_hbm, vemb_hbm,
             ids_v, rr_v, blocks_v, starts_v, ring_v, echunk_v, *sems):
    wid = lax.axis_index("s") * 2 + lax.axis_index("c")
    _gather_phase(utT_hbm, us_hbm, uemb_hbm, wid,
                  ids_v, rr_v, blocks_v, starts_v, ring_v, echunk_v, sems)
    _gather_phase(itT_hbm, vs_hbm, vemb_hbm, wid,
                  ids_v, rr_v, blocks_v, starts_v, ring_v, echunk_v, sems)


def _k2_body(uemb_hbm, vemb_hbm, iu_hbm, iv_hbm, out_hbm,
             iu_v, iv_v, u_rows, v_rows, partial_v, scores_v,
             sem_u, sem_v):
    wid = lax.axis_index("s") * 2 + lax.axis_index("c")
    base = wid * _BPW
    lane = lax.iota(jnp.int32, 16)
    nk = _BPW // 128

    def stage_idx(k):
        row0 = base + k * 128
        b = k & 1
        pltpu.sync_copy(iu_hbm.at[pl.ds(row0, 128)], iu_v.at[b])
        pltpu.sync_copy(iv_hbm.at[pl.ds(row0, 128)], iv_v.at[b])

    def fire_rows(k):
        b = k & 1
        pltpu.async_copy(uemb_hbm.at[iu_v.at[b]], u_rows.at[b], sem_u)
        pltpu.async_copy(vemb_hbm.at[iv_v.at[b]], v_rows.at[b], sem_v)

    def wait_rows(k):
        b = k & 1
        pltpu.make_async_copy(uemb_hbm.at[iu_v.at[b]], u_rows.at[b],
                              sem_u).wait()
        pltpu.make_async_copy(vemb_hbm.at[iv_v.at[b]], v_rows.at[b],
                              sem_v).wait()

    def compute(k):
        b = k & 1
        row0 = base + k * 128

        def row_body(r, carry):
            acc = jnp.zeros((16,), jnp.float32)
            for cc in range(_D // 16):
                gu = u_rows[b, r, pl.ds(cc * 16, 16)]
                gv = v_rows[b, r, pl.ds(cc * 16, 16)]
                acc = acc + gu * gv
            partial_v[pl.ds(r * 16, 16)] = acc
            return carry

        lax.fori_loop(0, 128, row_body, 0)

        def tile_body(t, carry):
            rowbase = (t * 16 + lane) * 16
            acc = jnp.zeros((16,), jnp.float32)
            for l in range(16):
                acc = acc + plsc.load_gather(partial_v, [rowbase + l])
            scores_v[pl.ds(t * 16, 16)] = acc
            return carry

        lax.fori_loop(0, 8, tile_body, 0)
        pltpu.sync_copy(scores_v, out_hbm.at[pl.ds(row0, 128)])

    stage_idx(0)
    fire_rows(0)
    for k in range(nk):
        if k + 1 < nk:
            stage_idx(k + 1)
            fire_rows(k + 1)
        wait_rows(k)
        compute(k)


def kernel(user_ids, item_ids, user_table, item_table):
    mesh = plsc.VectorSubcoreMesh(core_axis_name="c", subcore_axis_name="s")
    cp = pltpu.CompilerParams(
        needs_layout_passes=False, use_tc_tiling_on_sc=True)

    ui = user_ids.astype(jnp.int32)
    ii = item_ids.astype(jnp.int32)
    iota_b = jnp.arange(_B, dtype=jnp.int32)
    us, order_u = lax.sort_key_val(ui, iota_b)
    vs, order_v = lax.sort_key_val(ii, iota_b)
    inv_u = jnp.zeros((_B,), jnp.int32).at[order_u].set(iota_b)
    inv_v = jnp.zeros((_B,), jnp.int32).at[order_v].set(iota_b)

    k1 = functools.partial(
        pl.kernel, mesh=mesh, compiler_params=cp,
        out_type=(jax.ShapeDtypeStruct((_B * _EMBW,), jnp.float32),
                  jax.ShapeDtypeStruct((_B * _EMBW,), jnp.float32)),
        scratch_types=[
            pltpu.VMEM((_BPW,), jnp.int32),
            pltpu.VMEM((_BPW,), jnp.int32),
            pltpu.VMEM((_BPW + 48,), jnp.int32),
            pltpu.VMEM((_BPW + 48,), jnp.int32),
            pltpu.VMEM((_RING * 64, 128), jnp.float32),
            pltpu.VMEM((16 * _EMBW,), jnp.float32),
        ] + [pltpu.SemaphoreType.DMA] * _RING,
    )(_k1_body)
    uemb, vemb = k1(us, vs, user_table.T, item_table.T)

    uemb2 = uemb.reshape(_B, _EMBW)
    vemb2 = vemb.reshape(_B, _EMBW)

    k2 = functools.partial(
        pl.kernel, mesh=mesh, compiler_params=cp,
        out_type=jax.ShapeDtypeStruct((_B,), jnp.float32),
        scratch_types=[
            pltpu.VMEM((2, 128), jnp.int32),
            pltpu.VMEM((2, 128), jnp.int32),
            pltpu.VMEM((2, 128, _EMBW), jnp.float32),
            pltpu.VMEM((2, 128, _EMBW), jnp.float32),
            pltpu.VMEM((128 * 16,), jnp.float32),
            pltpu.VMEM((128,), jnp.float32),
            pltpu.SemaphoreType.DMA,
            pltpu.SemaphoreType.DMA,
        ],
    )(_k2_body)
    return k2(uemb2, vemb2, inv_u, inv_v)
